# baseline (speedup 1.0000x reference)
import functools

import jax
import jax.numpy as jnp
from jax import lax
from jax.experimental import pallas as pl
from jax.experimental.pallas import tpu as pltpu
from jax.experimental.pallas import tpu_sc as plsc

_NP = 10240
_STRIPE = _NP // 16
_K = 128
_NCH = 80
_NW = 32
_EPAD = _NW * _NCH * _K
_D = 128
_G = 64

_mesh = plsc.VectorSubcoreMesh(core_axis_name="c", subcore_axis_name="s")


def _sc_edge_scatter(table, sd4, zeros):
    @functools.partial(
        pl.kernel,
        out_type=jax.ShapeDtypeStruct((2, _NP, _D), jnp.float32),
        mesh=_mesh,
        scratch_types=[
            pltpu.VMEM((_K,), jnp.int32),
            pltpu.VMEM((_K,), jnp.int32),
            pltpu.VMEM((_K,), jnp.int32),
            pltpu.VMEM((_K,), jnp.int32),
            pltpu.VMEM((_K, _D), jnp.float32),
            pltpu.VMEM((_K, _D), jnp.float32),
            pltpu.VMEM_SHARED((_NP, _D), jnp.float32),
            pltpu.SemaphoreType.DMA,
            pltpu.SemaphoreType.DMA,
            pltpu.SemaphoreType.DMA,
            pltpu.SemaphoreType.DMA,
        ],
    )
    def k(table_h, sd_h, zeros_h, out_h, ss0, ss1, ds0, ds1, buf0, buf1,
          acc, gsem, ssem, isem, dsem):
        c = lax.axis_index("c")
        s = lax.axis_index("s")
        wid = c * 16 + s
        r0 = s * _STRIPE
        pltpu.sync_copy(zeros_h.at[pl.ds(r0, _STRIPE)],
                        acc.at[pl.ds(r0, _STRIPE)])
        sslots = (ss0, ss1)
        dslots = (ds0, ds1)
        bufs = (buf0, buf1)
        pltpu.sync_copy(sd_h.at[wid, 0, 0], ss0)
        pltpu.sync_copy(sd_h.at[wid, 0, 1], ds0)
        pltpu.async_copy(sd_h.at[wid, 1, 0], ss1, isem)
        plsc.subcore_barrier()
        pltpu.async_copy(table_h.at[ss0], buf0, gsem)

        def body(i, carry):
            for b in range(2):
                j = 2 * i + b
                jn1 = jnp.minimum(j + 1, _NCH - 1)
                jn2 = jnp.minimum(j + 2, _NCH - 1)
                pltpu.make_async_copy(table_h.at[ss0], bufs[b], gsem).wait()

                @pl.when(j > 0)
                def _():
                    pltpu.make_async_copy(bufs[0], acc.at[ds0], ssem).wait()
                    pltpu.make_async_copy(sd_h.at[wid, 0, 1], dslots[b],
                                          dsem).wait()

                pltpu.async_copy(bufs[b], acc.at[dslots[b]], ssem, add=True)
                pltpu.async_copy(sd_h.at[wid, jn1, 1], dslots[1 - b], dsem)
                pltpu.make_async_copy(sd_h.at[wid, 0, 0], sslots[1 - b],
                                      isem).wait()
                pltpu.async_copy(table_h.at[sslots[1 - b]], bufs[1 - b], gsem)
                pltpu.async_copy(sd_h.at[wid, jn2, 0], sslots[b], isem)
            return carry

        lax.fori_loop(0, _NCH // 2, body, 0)
        pltpu.make_async_copy(bufs[0], acc.at[ds0], ssem).wait()
        pltpu.make_async_copy(table_h.at[ss0], bufs[0], gsem).wait()
        pltpu.make_async_copy(sd_h.at[wid, 0, 0], ss0, isem).wait()
        pltpu.make_async_copy(sd_h.at[wid, 0, 1], ds0, dsem).wait()
        plsc.subcore_barrier()
        pltpu.sync_copy(acc.at[pl.ds(r0, _STRIPE)],
                        out_h.at[c, pl.ds(r0, _STRIPE)])

    return k(table, sd4, zeros)


def _sc_degree(dst2, zeros1d):
    @functools.partial(
        pl.kernel,
        out_type=jax.ShapeDtypeStruct((_NW, _NP), jnp.float32),
        mesh=_mesh,
        compiler_params=pltpu.CompilerParams(needs_layout_passes=False),
        scratch_types=[
            pltpu.VMEM((_NCH * _K,), jnp.int32),
            pltpu.VMEM((_NP,), jnp.float32),
        ],
    )
    def k(dst_h, zeros_h, out_h, dst_v, deg_v):
        c = lax.axis_index("c")
        s = lax.axis_index("s")
        wid = c * 16 + s
        pltpu.sync_copy(dst_h.at[wid], dst_v)
        pltpu.sync_copy(zeros_h, deg_v)
        ones = jnp.ones((16,), jnp.float32)

        def body(i, carry):
            d = dst_v[pl.ds(i * 16, 16)]
            plsc.addupdate_scatter(deg_v, [d], ones)
            return carry

        lax.fori_loop(0, (_NCH * _K) // 16, body, 0)
        pltpu.sync_copy(deg_v, out_h.at[wid])

    return k(dst2, zeros1d)


_BR = 1280


def _tc_scale(x, w, degp):
    def body(x_ref, w_ref, deg_ref, xs_ref, dinv_ref):
        deg = lax.dot_general(deg_ref[...], jnp.ones((_NW, 1), jnp.float32),
                              (((0,), (0,)), ((), ())),
                              preferred_element_type=jnp.float32)
        dinv = 1.0 / jnp.sqrt(deg + 1.0)
        xw = jnp.dot(x_ref[...], w_ref[...],
                     preferred_element_type=jnp.float32)
        xs_ref[...] = xw * dinv
        dinv_ref[...] = jnp.broadcast_to(dinv, (_BR, 8))

    return pl.pallas_call(
        body,
        grid=(_NP // _BR,),
        in_specs=[
            pl.BlockSpec((_BR, _D), lambda i: (i, 0)),
            pl.BlockSpec((_D, _D), lambda i: (0, 0)),
            pl.BlockSpec((_NW, _BR), lambda i: (0, i)),
        ],
        out_specs=(
            pl.BlockSpec((_BR, _D), lambda i: (i, 0)),
            pl.BlockSpec((_BR, 8), lambda i: (i, 0)),
        ),
        out_shape=(
            jax.ShapeDtypeStruct((_NP, _D), jnp.float32),
            jax.ShapeDtypeStruct((_NP, 8), jnp.float32),
        ),
    )(x, w, degp)


def _tc_mid(accp, xs, dinv8, b, w):
    def body(acc_ref, xs_ref, dinv_ref, b_ref, w_ref, o_ref):
        dinv = dinv_ref[:, 0:1]
        h = jnp.maximum(
            (acc_ref[0] + acc_ref[1] + xs_ref[...]) * dinv + b_ref[...], 0.0)
        o_ref[...] = jnp.dot(h, w_ref[...],
                             preferred_element_type=jnp.float32) * dinv

    return pl.pallas_call(
        body,
        grid=(_NP // _BR,),
        in_specs=[
            pl.BlockSpec((2, _BR, _D), lambda i: (0, i, 0)),
            pl.BlockSpec((_BR, _D), lambda i: (i, 0)),
            pl.BlockSpec((_BR, 8), lambda i: (i, 0)),
            pl.BlockSpec((1, _D), lambda i: (0, 0)),
            pl.BlockSpec((_D, _D), lambda i: (0, 0)),
        ],
        out_specs=pl.BlockSpec((_BR, _D), lambda i: (i, 0)),
        out_shape=jax.ShapeDtypeStruct((_NP, _D), jnp.float32),
    )(accp, xs, dinv8, b, w)


def _tc_head(accp, xs, dinv8, b, batch_p, fcW1, fcb1, fcW2, fcb2, fcW3, fcb3,
             fcW4, fcb4):
    d_out = fcW4.shape[1]

    def body(acc_ref, xs_ref, dinv_ref, b_ref, batch_ref, w1_ref, c1_ref,
             w2_ref, c2_ref, w3_ref, c3_ref, w4_ref, c4_ref, f_ref, y_ref,
             sums_ref, cnts_ref):
        i = pl.program_id(0)

        @pl.when(i == 0)
        def _():
            sums_ref[...] = jnp.zeros((_G, _D), jnp.float32)
            cnts_ref[...] = jnp.zeros((_G, 8), jnp.float32)

        dinv = dinv_ref[:, 0:1]
        h = (acc_ref[0] + acc_ref[1] + xs_ref[...]) * dinv + b_ref[...]
        ids = batch_ref[...]
        onehot = (ids == lax.broadcasted_iota(jnp.int32, (_BR, _G), 1)
                  ).astype(jnp.float32)
        dims = (((0,), (0,)), ((), ()))
        sums_ref[...] += lax.dot_general(onehot, h, dims,
                                         preferred_element_type=jnp.float32)
        cnts_ref[...] += lax.dot_general(
            onehot, jnp.ones((_BR, 8), jnp.float32), dims,
            preferred_element_type=jnp.float32)

        @pl.when(i == _NP // _BR - 1)
        def _():
            f = sums_ref[...] / jnp.maximum(cnts_ref[:, 0:1], 1.0)
            y = jnp.maximum(jnp.dot(f, w1_ref[...],
                                    preferred_element_type=jnp.float32)
                            + c1_ref[...], 0.0)
            y = jnp.maximum(jnp.dot(y, w2_ref[...],
                                    preferred_element_type=jnp.float32)
                            + c2_ref[...], 0.0)
            y = jnp.maximum(jnp.dot(y, w3_ref[...],
                                    preferred_element_type=jnp.float32)
                            + c3_ref[...], 0.0)
            y = jnp.dot(y, w4_ref[...],
                        preferred_element_type=jnp.float32) + c4_ref[...]
            f_ref[...] = f
            y_ref[...] = y

    return pl.pallas_call(
        body,
        grid=(_NP // _BR,),
        in_specs=[
            pl.BlockSpec((2, _BR, _D), lambda i: (0, i, 0)),
            pl.BlockSpec((_BR, _D), lambda i: (i, 0)),
            pl.BlockSpec((_BR, 8), lambda i: (i, 0)),
            pl.BlockSpec((1, _D), lambda i: (0, 0)),
            pl.BlockSpec((_BR, 1), lambda i: (i, 0)),
            pl.BlockSpec(fcW1.shape, lambda i: (0, 0)),
            pl.BlockSpec((1, fcW1.shape[1]), lambda i: (0, 0)),
            pl.BlockSpec(fcW2.shape, lambda i: (0, 0)),
            pl.BlockSpec((1, fcW2.shape[1]), lambda i: (0, 0)),
            pl.BlockSpec(fcW3.shape, lambda i: (0, 0)),
            pl.BlockSpec((1, fcW3.shape[1]), lambda i: (0, 0)),
            pl.BlockSpec(fcW4.shape, lambda i: (0, 0)),
            pl.BlockSpec((1, d_out), lambda i: (0, 0)),
        ],
        out_specs=(
            pl.BlockSpec((_G, _D), lambda i: (0, 0)),
            pl.BlockSpec((_G, d_out), lambda i: (0, 0)),
        ),
        out_shape=(
            jax.ShapeDtypeStruct((_G, _D), jnp.float32),
            jax.ShapeDtypeStruct((_G, d_out), jnp.float32),
        ),
        scratch_shapes=[
            pltpu.VMEM((_G, _D), jnp.float32),
            pltpu.VMEM((_G, 8), jnp.float32),
        ],
    )(accp, xs, dinv8, b, batch_p, fcW1, fcb1, fcW2, fcb2, fcW3, fcb3, fcW4,
      fcb4)


def kernel(x, edge_index, batch, W1, b1, W2, b2, W3, b3, fcW1, fcb1, fcW2,
           fcb2, fcW3, fcb3, fcW4, fcb4):
    n = x.shape[0]
    e = edge_index.shape[1]
    fill = n + jnp.arange(_EPAD - e, dtype=jnp.int32) % (_NP - n)
    ei_pad = jnp.concatenate(
        [edge_index, jnp.broadcast_to(fill, (2, _EPAD - e))], axis=1)
    sd4 = ei_pad.reshape(2, _NW, _NCH, _K).transpose(1, 2, 0, 3)
    dst2 = sd4[:, :, 1, :].reshape(_NW, _NCH * _K)
    x_p = jnp.pad(x, ((0, _NP - n), (0, 0)))
    batch_p = jnp.concatenate(
        [batch, jnp.full((_NP - n,), -1, jnp.int32)]).reshape(_NP, 1)
    zeros = jnp.zeros((_NP, _D), jnp.float32)
    zeros1d = jnp.zeros((_NP,), jnp.float32)

    degp = _sc_degree(dst2, zeros1d)
    xs1, dinv8 = _tc_scale(x_p, W1, degp)
    acc1 = _sc_edge_scatter(xs1, sd4, zeros)
    xs2 = _tc_mid(acc1, xs1, dinv8, b1.reshape(1, -1), W2)
    acc2 = _sc_edge_scatter(xs2, sd4, zeros)
    xs3 = _tc_mid(acc2, xs2, dinv8, b2.reshape(1, -1), W3)
    acc3 = _sc_edge_scatter(xs3, sd4, zeros)
    f, y = _tc_head(acc3, xs3, dinv8, b3.reshape(1, -1), batch_p,
                    fcW1, fcb1.reshape(1, -1), fcW2, fcb2.reshape(1, -1),
                    fcW3, fcb3.reshape(1, -1), fcW4, fcb4.reshape(1, -1))
    return (f, y)


# speedup vs baseline: 1.0122x; 1.0122x over previous
import functools

import jax
import jax.numpy as jnp
from jax import lax
from jax.experimental import pallas as pl
from jax.experimental.pallas import tpu as pltpu
from jax.experimental.pallas import tpu_sc as plsc

_NP = 10240
_STRIPE = _NP // 16
_K = 128
_NCH = 80
_NW = 32
_EPAD = _NW * _NCH * _K
_ZR = 64
_D = 128
_G = 64

_mesh = plsc.VectorSubcoreMesh(core_axis_name="c", subcore_axis_name="s")


def _sc_edge_scatter(table, sd4, zeros):
    @functools.partial(
        pl.kernel,
        out_type=jax.ShapeDtypeStruct((2, _NP, _D), jnp.float32),
        mesh=_mesh,
        scratch_types=[
            pltpu.VMEM((2, 2, _K), jnp.int32),
            pltpu.VMEM((_K, _D), jnp.float32),
            pltpu.VMEM((_K, _D), jnp.float32),
            pltpu.VMEM((_ZR, _D), jnp.float32),
            pltpu.VMEM_SHARED((_NP, _D), jnp.float32),
            pltpu.SemaphoreType.DMA,
            pltpu.SemaphoreType.DMA,
        ],
    )
    def k(table_h, sd_h, zeros_h, out_h, idx_v, buf0, buf1, zbuf, acc,
          gsem, isem):
        c = lax.axis_index("c")
        s = lax.axis_index("s")
        wid = c * 16 + s
        r0 = s * _STRIPE
        pltpu.sync_copy(zeros_h, zbuf)
        for z in range(_STRIPE // _ZR):
            pltpu.sync_copy(zbuf, acc.at[pl.ds(r0 + z * _ZR, _ZR)])
        pltpu.sync_copy(sd_h.at[wid, 0], idx_v.at[0])
        pltpu.async_copy(sd_h.at[wid, 1], idx_v.at[1], isem)
        plsc.subcore_barrier()

        bufs = (buf0, buf1)
        pltpu.async_copy(table_h.at[idx_v.at[0, 0]], buf0, gsem)

        def body(i, carry):
            for b in range(2):
                j = 2 * i + b
                jn1 = jnp.minimum(j + 1, _NCH - 1)
                jn2 = jnp.minimum(j + 2, _NCH - 1)
                pltpu.make_async_copy(table_h.at[idx_v.at[0, 0]], bufs[b],
                                      gsem).wait()
                pltpu.make_async_copy(sd_h.at[wid, 0], idx_v.at[0],
                                      isem).wait()
                pltpu.async_copy(table_h.at[idx_v.at[1 - b, 0]], bufs[1 - b],
                                 gsem)
                pltpu.sync_copy(bufs[b], acc.at[idx_v.at[b, 1]], add=True)
                pltpu.async_copy(sd_h.at[wid, jn2], idx_v.at[b], isem)
            return carry

        lax.fori_loop(0, _NCH // 2, body, 0)
        pltpu.make_async_copy(table_h.at[idx_v.at[0, 0]], bufs[0],
                              gsem).wait()
        pltpu.make_async_copy(sd_h.at[wid, 0], idx_v.at[0], isem).wait()
        plsc.subcore_barrier()
        pltpu.sync_copy(acc.at[pl.ds(r0, _STRIPE)],
                        out_h.at[c, pl.ds(r0, _STRIPE)])

    return k(table, sd4, zeros)


def _sc_degree(dst2, zeros1d):
    @functools.partial(
        pl.kernel,
        out_type=jax.ShapeDtypeStruct((_NW, _NP), jnp.float32),
        mesh=_mesh,
        compiler_params=pltpu.CompilerParams(needs_layout_passes=False),
        scratch_types=[
            pltpu.VMEM((_NCH * _K,), jnp.int32),
            pltpu.VMEM((_NP,), jnp.float32),
        ],
    )
    def k(dst_h, zeros_h, out_h, dst_v, deg_v):
        c = lax.axis_index("c")
        s = lax.axis_index("s")
        wid = c * 16 + s
        pltpu.sync_copy(dst_h.at[wid], dst_v)
        pltpu.sync_copy(zeros_h, deg_v)
        ones = jnp.ones((16,), jnp.float32)

        def body(i, carry):
            d = dst_v[pl.ds(i * 16, 16)]
            plsc.addupdate_scatter(deg_v, [d], ones)
            return carry

        lax.fori_loop(0, (_NCH * _K) // 16, body, 0)
        pltpu.sync_copy(deg_v, out_h.at[wid])

    return k(dst2, zeros1d)


_BR = 1280


def _tc_scale(x, w, degp):
    def body(x_ref, w_ref, deg_ref, xs_ref, dinv_ref):
        deg = lax.dot_general(deg_ref[...], jnp.ones((_NW, 1), jnp.float32),
                              (((0,), (0,)), ((), ())),
                              preferred_element_type=jnp.float32)
        dinv = 1.0 / jnp.sqrt(deg + 1.0)
        xw = jnp.dot(x_ref[...], w_ref[...],
                     preferred_element_type=jnp.float32)
        xs_ref[...] = xw * dinv
        dinv_ref[...] = jnp.broadcast_to(dinv, (_BR, 8))

    return pl.pallas_call(
        body,
        grid=(_NP // _BR,),
        in_specs=[
            pl.BlockSpec((_BR, _D), lambda i: (i, 0)),
            pl.BlockSpec((_D, _D), lambda i: (0, 0)),
            pl.BlockSpec((_NW, _BR), lambda i: (0, i)),
        ],
        out_specs=(
            pl.BlockSpec((_BR, _D), lambda i: (i, 0)),
            pl.BlockSpec((_BR, 8), lambda i: (i, 0)),
        ),
        out_shape=(
            jax.ShapeDtypeStruct((_NP, _D), jnp.float32),
            jax.ShapeDtypeStruct((_NP, 8), jnp.float32),
        ),
    )(x, w, degp)


def _tc_mid(accp, xs, dinv8, b, w):
    def body(acc_ref, xs_ref, dinv_ref, b_ref, w_ref, o_ref):
        dinv = dinv_ref[:, 0:1]
        h = jnp.maximum(
            (acc_ref[0] + acc_ref[1] + xs_ref[...]) * dinv + b_ref[...], 0.0)
        o_ref[...] = jnp.dot(h, w_ref[...],
                             preferred_element_type=jnp.float32) * dinv

    return pl.pallas_call(
        body,
        grid=(_NP // _BR,),
        in_specs=[
            pl.BlockSpec((2, _BR, _D), lambda i: (0, i, 0)),
            pl.BlockSpec((_BR, _D), lambda i: (i, 0)),
            pl.BlockSpec((_BR, 8), lambda i: (i, 0)),
            pl.BlockSpec((1, _D), lambda i: (0, 0)),
            pl.BlockSpec((_D, _D), lambda i: (0, 0)),
        ],
        out_specs=pl.BlockSpec((_BR, _D), lambda i: (i, 0)),
        out_shape=jax.ShapeDtypeStruct((_NP, _D), jnp.float32),
    )(accp, xs, dinv8, b, w)


def _tc_head(accp, xs, dinv8, b, batch_p, fcW1, fcb1, fcW2, fcb2, fcW3, fcb3,
             fcW4, fcb4):
    d_out = fcW4.shape[1]

    def body(acc_ref, xs_ref, dinv_ref, b_ref, batch_ref, w1_ref, c1_ref,
             w2_ref, c2_ref, w3_ref, c3_ref, w4_ref, c4_ref, f_ref, y_ref,
             sums_ref, cnts_ref):
        i = pl.program_id(0)

        @pl.when(i == 0)
        def _():
            sums_ref[...] = jnp.zeros((_G, _D), jnp.float32)
            cnts_ref[...] = jnp.zeros((_G, 8), jnp.float32)

        dinv = dinv_ref[:, 0:1]
        h = (acc_ref[0] + acc_ref[1] + xs_ref[...]) * dinv + b_ref[...]
        ids = batch_ref[...]
        onehot = (ids == lax.broadcasted_iota(jnp.int32, (_BR, _G), 1)
                  ).astype(jnp.float32)
        dims = (((0,), (0,)), ((), ()))
        sums_ref[...] += lax.dot_general(onehot, h, dims,
                                         preferred_element_type=jnp.float32)
        cnts_ref[...] += lax.dot_general(
            onehot, jnp.ones((_BR, 8), jnp.float32), dims,
            preferred_element_type=jnp.float32)

        @pl.when(i == _NP // _BR - 1)
        def _():
            f = sums_ref[...] / jnp.maximum(cnts_ref[:, 0:1], 1.0)
            y = jnp.maximum(jnp.dot(f, w1_ref[...],
                                    preferred_element_type=jnp.float32)
                            + c1_ref[...], 0.0)
            y = jnp.maximum(jnp.dot(y, w2_ref[...],
                                    preferred_element_type=jnp.float32)
                            + c2_ref[...], 0.0)
            y = jnp.maximum(jnp.dot(y, w3_ref[...],
                                    preferred_element_type=jnp.float32)
                            + c3_ref[...], 0.0)
            y = jnp.dot(y, w4_ref[...],
                        preferred_element_type=jnp.float32) + c4_ref[...]
            f_ref[...] = f
            y_ref[...] = y

    return pl.pallas_call(
        body,
        grid=(_NP // _BR,),
        in_specs=[
            pl.BlockSpec((2, _BR, _D), lambda i: (0, i, 0)),
            pl.BlockSpec((_BR, _D), lambda i: (i, 0)),
            pl.BlockSpec((_BR, 8), lambda i: (i, 0)),
            pl.BlockSpec((1, _D), lambda i: (0, 0)),
            pl.BlockSpec((_BR, 1), lambda i: (i, 0)),
            pl.BlockSpec(fcW1.shape, lambda i: (0, 0)),
            pl.BlockSpec((1, fcW1.shape[1]), lambda i: (0, 0)),
            pl.BlockSpec(fcW2.shape, lambda i: (0, 0)),
            pl.BlockSpec((1, fcW2.shape[1]), lambda i: (0, 0)),
            pl.BlockSpec(fcW3.shape, lambda i: (0, 0)),
            pl.BlockSpec((1, fcW3.shape[1]), lambda i: (0, 0)),
            pl.BlockSpec(fcW4.shape, lambda i: (0, 0)),
            pl.BlockSpec((1, d_out), lambda i: (0, 0)),
        ],
        out_specs=(
            pl.BlockSpec((_G, _D), lambda i: (0, 0)),
            pl.BlockSpec((_G, d_out), lambda i: (0, 0)),
        ),
        out_shape=(
            jax.ShapeDtypeStruct((_G, _D), jnp.float32),
            jax.ShapeDtypeStruct((_G, d_out), jnp.float32),
        ),
        scratch_shapes=[
            pltpu.VMEM((_G, _D), jnp.float32),
            pltpu.VMEM((_G, 8), jnp.float32),
        ],
    )(accp, xs, dinv8, b, batch_p, fcW1, fcb1, fcW2, fcb2, fcW3, fcb3, fcW4,
      fcb4)


def kernel(x, edge_index, batch, W1, b1, W2, b2, W3, b3, fcW1, fcb1, fcW2,
           fcb2, fcW3, fcb3, fcW4, fcb4):
    n = x.shape[0]
    e = edge_index.shape[1]
    fill = n + jnp.arange(_EPAD - e, dtype=jnp.int32) % (_NP - n)
    ei_pad = jnp.concatenate(
        [edge_index, jnp.broadcast_to(fill, (2, _EPAD - e))], axis=1)
    sd4 = ei_pad.reshape(2, _NW, _NCH, _K).transpose(1, 2, 0, 3)
    dst2 = sd4[:, :, 1, :].reshape(_NW, _NCH * _K)
    x_p = jnp.pad(x, ((0, _NP - n), (0, 0)))
    batch_p = jnp.concatenate(
        [batch, jnp.full((_NP - n,), -1, jnp.int32)]).reshape(_NP, 1)
    zeros = jnp.zeros((_ZR, _D), jnp.float32)
    zeros1d = jnp.zeros((_NP,), jnp.float32)

    degp = _sc_degree(dst2, zeros1d)
    xs1, dinv8 = _tc_scale(x_p, W1, degp)
    acc1 = _sc_edge_scatter(xs1, sd4, zeros)
    xs2 = _tc_mid(acc1, xs1, dinv8, b1.reshape(1, -1), W2)
    acc2 = _sc_edge_scatter(xs2, sd4, zeros)
    xs3 = _tc_mid(acc2, xs2, dinv8, b2.reshape(1, -1), W3)
    acc3 = _sc_edge_scatter(xs3, sd4, zeros)
    f, y = _tc_head(acc3, xs3, dinv8, b3.reshape(1, -1), batch_p,
                    fcW1, fcb1.reshape(1, -1), fcW2, fcb2.reshape(1, -1),
                    fcW3, fcb3.reshape(1, -1), fcW4, fcb4.reshape(1, -1))
    return (f, y)
